# baseline (speedup 1.0000x reference)
import functools

import jax
import jax.numpy as jnp
from jax import lax
from jax.experimental import pallas as pl
from jax.experimental.pallas import tpu as pltpu
from jax.experimental.pallas import tpu_sc as plsc

N = 100000
K = 128
CH = 160
CHK = CH * K
NCH = N // CH
NW = 32
MAXC = -(-NCH // NW)
NS = 4

_mesh = plsc.VectorSubcoreMesh(core_axis_name="c", subcore_axis_name="s")


@functools.partial(
    pl.kernel,
    mesh=_mesh,
    out_type=jax.ShapeDtypeStruct((N * K,), jnp.float32),
    scratch_types=(
        [pltpu.VMEM((CHK,), jnp.float32)]
        + [pltpu.VMEM((CH,), jnp.int32) for _ in range(NS)]
        + [pltpu.VMEM((128,), jnp.int32) for _ in range(NS)]
        + [pltpu.VMEM((32,), jnp.int32) for _ in range(NS)]
        + [pltpu.VMEM((128,), jnp.float32),
           pltpu.VMEM((32,), jnp.float32)]
        + [pltpu.SemaphoreType.DMA for _ in range(3 * NS)]
    ),
    compiler_params=pltpu.CompilerParams(needs_layout_passes=False),
)
def _one_hot_sc(atom_hbm, out_hbm, zb, *scratch):
    sidx = scratch[:NS]
    fas = scratch[NS:2 * NS]
    fbs = scratch[2 * NS:3 * NS]
    onesA, onesB = scratch[3 * NS:3 * NS + 2]
    sems = scratch[3 * NS + 2:]
    zsems = sems[:NS]
    ssems = sems[NS:2 * NS]
    isems = sems[2 * NS:3 * NS]

    info = plsc.get_sparse_core_info()
    wid = lax.axis_index("s") * info.num_cores + lax.axis_index("c")

    zvec = jnp.zeros((16,), jnp.float32)
    ovec = jnp.ones((16,), jnp.float32)
    row_off = lax.iota(jnp.int32, 16) * K

    n_mine = (NCH - wid + NW - 1) // NW

    def _prefetch_idx(m, s):
        chunk = wid + m * NW
        pltpu.async_copy(atom_hbm.at[pl.ds(chunk * CH, CH)], sidx[s],
                         isems[s])

    def _issue_zero(m, s):
        chunk = wid + m * NW
        pltpu.async_copy(zb, out_hbm.at[pl.ds(chunk * CHK, CHK)], zsems[s])

    def _wait_scatters(s):
        pltpu.make_async_copy(onesA, out_hbm.at[fas[s]], ssems[s]).wait()
        pltpu.make_async_copy(onesB, out_hbm.at[fbs[s]], ssems[s]).wait()

    def _body(m, s, first):
        if not first:
            _wait_scatters(s)

        @pl.when(m + NS - 1 < n_mine)
        def _():
            _issue_zero(m + NS - 1, (s + NS - 1) % NS)

        chunk = wid + m * NW
        base = chunk * CHK
        pltpu.make_async_copy(atom_hbm.at[pl.ds(0, CH)], sidx[s],
                              isems[s]).wait()
        sx, fa, fb = sidx[s], fas[s], fbs[s]

        def fidx_body(g, carry):
            flat = row_off + (base + g * (16 * K)) + sx[pl.ds(g * 16, 16)]
            fa[pl.ds(g * 16, 16)] = flat
            return carry

        lax.fori_loop(0, 8, fidx_body, 0, unroll=4)
        for g in (8, 9):
            flat = row_off + (base + g * (16 * K)) + sx[pl.ds(g * 16, 16)]
            fb[pl.ds((g - 8) * 16, 16)] = flat

        @pl.when(m + NS < n_mine)
        def _():
            _prefetch_idx(m + NS, s)

        pltpu.make_async_copy(zb, out_hbm.at[pl.ds(0, CHK)], zsems[s]).wait()
        pltpu.async_copy(onesA, out_hbm.at[fas[s]], ssems[s])
        pltpu.async_copy(onesB, out_hbm.at[fbs[s]], ssems[s])

    for s in range(NS):
        _prefetch_idx(s, s)

    def _zero_body(i, carry):
        zb[pl.ds(i * 16, 16)] = zvec
        return carry

    lax.fori_loop(0, CHK // 16, _zero_body, 0, unroll=8)
    for i in range(8):
        onesA[pl.ds(i * 16, 16)] = ovec
    for i in range(2):
        onesB[pl.ds(i * 16, 16)] = ovec

    for m in range(NS - 1):
        _issue_zero(m, m % NS)

    for m in range(NS):
        _body(m, m, True)

    def _quad_body(q, carry):
        for j in range(NS):
            m = NS * q + j

            @pl.when(m < n_mine)
            def _():
                _body(m, j, False)
        return carry

    lax.fori_loop(1, -(-MAXC // NS), _quad_body, 0)

    for s in range(NS):
        _wait_scatters(s)


def kernel(atom_type, pos):
    del pos
    out = _one_hot_sc(atom_type.astype(jnp.int32))
    return out.reshape(N, K)


# speedup vs baseline: 3.4740x; 3.4740x over previous
import functools

import jax
import jax.numpy as jnp
from jax import lax
from jax.experimental import pallas as pl
from jax.experimental.pallas import tpu as pltpu
from jax.experimental.pallas import tpu_sc as plsc

N = 100000
K = 128
CH = 160
CHK = CH * K
NCH = N // CH
NW = 32
GROUPS = CH // 16
NB = 2
MAXC = -(-NCH // NW)

_mesh = plsc.VectorSubcoreMesh(core_axis_name="c", subcore_axis_name="s")


@functools.partial(
    pl.kernel,
    mesh=_mesh,
    out_type=jax.ShapeDtypeStruct((N * K,), jnp.float32),
    scratch_types=(
        [pltpu.VMEM((CHK,), jnp.float32) for _ in range(NB)]
        + [pltpu.VMEM((CH,), jnp.int32) for _ in range(2 * NB)]
        + [pltpu.SemaphoreType.DMA for _ in range(2 * NB)]
    ),
    compiler_params=pltpu.CompilerParams(needs_layout_passes=False),
)
def _one_hot_sc(atom_hbm, out_hbm, *scratch):
    bufs = scratch[:NB]
    idxs = scratch[NB:2 * NB]
    fis = scratch[2 * NB:3 * NB]
    outsems = scratch[3 * NB:4 * NB]
    idxsems = scratch[4 * NB:5 * NB]

    info = plsc.get_sparse_core_info()
    wid = lax.axis_index("s") * info.num_cores + lax.axis_index("c")

    zvec = jnp.zeros((16,), jnp.float32)
    ovec = jnp.ones((16,), jnp.float32)
    row_off = lax.iota(jnp.int32, 16) * K

    n_mine = (NCH - wid + NW - 1) // NW

    def _prefetch_idx(ci, b):
        chunk = wid + ci * NW
        pltpu.async_copy(atom_hbm.at[pl.ds(chunk * CH, CH)],
                         idxs[b], idxsems[b])

    def _fill(b):
        buf, idx_v, fi = bufs[b], idxs[b], fis[b]

        def body(g, carry):
            base = row_off + g * (16 * K)
            flat = base + idx_v[pl.ds(g * 16, 16)]
            fi[pl.ds(g * 16, 16)] = flat
            plsc.store_scatter(buf, [flat], ovec)
            return carry

        lax.fori_loop(0, GROUPS, body, 0, unroll=GROUPS)

    def _clear(b):
        buf, fi = bufs[b], fis[b]

        def body(g, carry):
            plsc.store_scatter(buf, [fi[pl.ds(g * 16, 16)]], zvec)
            return carry

        lax.fori_loop(0, GROUPS, body, 0, unroll=GROUPS)

    for b in range(NB):
        _prefetch_idx(b, b)

    for b in range(NB):
        buf = bufs[b]

        def _zero_body(i, carry):
            buf[pl.ds(i * 16, 16)] = zvec
            return carry

        lax.fori_loop(0, CHK // 16, _zero_body, 0, unroll=16)
        pltpu.make_async_copy(atom_hbm.at[pl.ds(0, CH)], idxs[b],
                              idxsems[b]).wait()
        _fill(b)
        chunk = wid + b * NW
        pltpu.async_copy(bufs[b], out_hbm.at[pl.ds(chunk * CHK, CHK)],
                         outsems[b])
        _prefetch_idx(b + NB, b)

    def _group_body(i2, carry):
        for b in range(NB):
            ci = NB * i2 + b

            @pl.when(ci < n_mine)
            def _():
                pltpu.make_async_copy(bufs[b], out_hbm.at[pl.ds(0, CHK)],
                                      outsems[b]).wait()
                _clear(b)
                pltpu.make_async_copy(atom_hbm.at[pl.ds(0, CH)], idxs[b],
                                      idxsems[b]).wait()
                _fill(b)
                chunk = wid + ci * NW
                pltpu.async_copy(bufs[b],
                                 out_hbm.at[pl.ds(chunk * CHK, CHK)],
                                 outsems[b])

                @pl.when(ci + NB < n_mine)
                def _():
                    _prefetch_idx(ci + NB, b)
        return carry

    lax.fori_loop(1, (MAXC + NB - 1) // NB, _group_body, 0)

    for b in range(NB):
        pltpu.make_async_copy(bufs[b], out_hbm.at[pl.ds(0, CHK)],
                              outsems[b]).wait()


def kernel(atom_type, pos):
    del pos
    out = _one_hot_sc(atom_type.astype(jnp.int32))
    return out.reshape(N, K)
